# baseline (speedup 1.0000x reference)
import jax
import jax.numpy as jnp
from jax import lax
from jax.experimental import pallas as pl
from jax.experimental.pallas import tpu as pltpu
from jax.experimental.pallas import tpu_sc as plsc

L, B, S, D = 13, 4, 2048, 768
NROWS = 3 * B
NLANES = 16
LAST_BASE = (L - 1) * B * S


def _sc_gather(table, idx1, idx2):
    mesh = plsc.VectorSubcoreMesh(
        core_axis_name="c", subcore_axis_name="s", num_cores=1)

    @pl.kernel(
        mesh=mesh,
        out_type=jax.ShapeDtypeStruct((NROWS, D), jnp.float32),
        scratch_types=[
            pltpu.VMEM((NLANES,), jnp.int32),
            pltpu.VMEM((NLANES,), jnp.int32),
            pltpu.VMEM((NROWS, D), jnp.float32),
            pltpu.SemaphoreType.DMA,
        ],
    )
    def k(table_hbm, idx1_hbm, idx2_hbm, out_hbm,
          idx_v, ridx_v, rows_v, sem):
        is_w0 = lax.axis_index("s") == 0

        @pl.when(is_w0)
        def _():
            cp_1 = pltpu.async_copy(idx1_hbm, idx_v.at[pl.ds(0, B)], sem)
            cp_2 = pltpu.async_copy(idx2_hbm, idx_v.at[pl.ds(8, B)], sem)
            lane = lax.iota(jnp.int32, NLANES)
            r = jnp.minimum(lane, NROWS - 1)
            b = (r * 21846) >> 16
            j = r - 3 * b
            src = jnp.where(j == 1, b, jnp.where(j == 2, 8 + b, 0))
            msk = jnp.where(j == 0, 0, 1)
            base = LAST_BASE + b * S
            cp_1.wait()
            cp_2.wait()
            tokens = lax.gather(
                idx_v[...], src[:, None],
                lax.GatherDimensionNumbers(
                    offset_dims=(), collapsed_slice_dims=(0,),
                    start_index_map=(0,)),
                slice_sizes=(1,),
                mode=lax.GatherScatterMode.PROMISE_IN_BOUNDS)
            ridx_v[...] = base + tokens * msk
            pltpu.async_copy(
                table_hbm.at[ridx_v.at[pl.ds(0, NROWS)]], rows_v, sem).wait()
            pltpu.sync_copy(rows_v, out_hbm)

    return k(table, idx1, idx2)


def kernel(input, idx1, idx2):
    table = input.reshape(L * B * S, D)
    out = _sc_gather(table, idx1, idx2)
    return out.reshape(B, 3 * D)
